# baseline (speedup 1.0000x reference)
import functools

import jax
import jax.numpy as jnp
from jax import lax
from jax.experimental import pallas as pl
from jax.experimental.pallas import tpu as pltpu
from jax.experimental.pallas import tpu_sc as plsc

N = 10000
NPAD = 10240
E = 320000
D = 128
G = 64
NC = 2
NS = 16
EK = 80
NCHUNK = 125
WE = E // (NC * NS)
RPS = NPAD // NS
EDGES_PER_SUB = E // (NC * NS)
NB = 10
BN = NPAD // NB


def _sc_mesh():
    return plsc.VectorSubcoreMesh(core_axis_name="c", subcore_axis_name="s")


def _deg_body(ei_hbm, out_hbm, didx_v, hist_v):
    c = lax.axis_index("c")
    s = lax.axis_index("s")
    zeros16 = jnp.zeros((16,), jnp.float32)

    def zstep(i, carry):
        hist_v[pl.ds(i * 16, 16)] = zeros16
        return carry

    lax.fori_loop(0, NPAD // 16, zstep, 0)
    base = c * (E // NC) + s * EDGES_PER_SUB
    pltpu.sync_copy(ei_hbm.at[pl.ds(E + base, EDGES_PER_SUB)], didx_v)
    ones16 = jnp.ones((16,), jnp.float32)

    def step(i, carry):
        idx = didx_v[pl.ds(i * 16, 16)]
        plsc.addupdate_scatter(hist_v, [idx], ones16)
        return carry

    lax.fori_loop(0, EDGES_PER_SUB // 16, step, 0)
    pltpu.sync_copy(hist_v, out_hbm.at[c, s])


def _sc_degree(ei):
    return pl.kernel(
        _deg_body,
        out_type=jax.ShapeDtypeStruct((NC, NS, NPAD), jnp.float32),
        mesh=_sc_mesh(),
        compiler_params=pltpu.CompilerParams(needs_layout_passes=False),
        scratch_types=[
            pltpu.VMEM((EDGES_PER_SUB,), jnp.int32),
            pltpu.VMEM((NPAD,), jnp.float32),
        ],
    )(ei)


def _scatter_body(g_hbm, ei_hbm, out_hbm,
                  si0, si1, si2, si3, di0, di1, di2, di3,
                  r0_, r1_, r2_, r3_,
                  is0, is1, is2, is3, gs0, gs1, gs2, gs3,
                  ss0, ss1, ss2, ss3, acc_sh):
    c = lax.axis_index("c")
    s = lax.axis_index("s")
    w = c * NS + s
    r0 = s * RPS
    pltpu.sync_copy(g_hbm.at[pl.ds(r0, RPS)], acc_sh.at[pl.ds(r0, RPS)])
    plsc.subcore_barrier()
    base = w * WE

    sidx = (si0, si1, si2, si3)
    didx = (di0, di1, di2, di3)
    rows = (r0_, r1_, r2_, r3_)
    isem = (is0, is1, is2, is3)
    gsem = (gs0, gs1, gs2, gs3)
    ssem = (ss0, ss1, ss2, ss3)

    def prefetch(j, b):
        pltpu.async_copy(ei_hbm.at[pl.ds(base + j * EK, EK)], sidx[b],
                         isem[b])
        pltpu.async_copy(ei_hbm.at[pl.ds(E + base + j * EK, EK)], didx[b],
                         isem[b])

    def iwait(b):
        pltpu.make_async_copy(ei_hbm.at[pl.ds(0, EK)], sidx[b],
                              isem[b]).wait()
        pltpu.make_async_copy(ei_hbm.at[pl.ds(0, EK)], didx[b],
                              isem[b]).wait()

    def gissue(j, b):
        pltpu.async_copy(g_hbm.at[sidx[b]], rows[b], gsem[b])

    def gwait(b):
        pltpu.make_async_copy(g_hbm.at[sidx[b]], rows[b], gsem[b]).wait()

    def sissue(j, b):
        pltpu.async_copy(rows[b], acc_sh.at[didx[b]], ssem[b], add=True)

    def swait(b):
        pltpu.make_async_copy(rows[b], acc_sh.at[didx[b]], ssem[b]).wait()

    prefetch(0, 0)
    prefetch(1, 1)
    iwait(0)
    gissue(0, 0)
    iwait(1)
    gissue(1, 1)
    gwait(0)
    sissue(0, 0)
    prefetch(2, 2)
    iwait(2)
    gissue(2, 2)
    gwait(1)
    sissue(1, 1)
    prefetch(3, 3)

    def quad(k, carry):
        for jj in range(4):
            j = 4 * k + jj + 2
            b = (jj + 2) % 4
            swait((jj) % 4)
            iwait((jj + 3) % 4)
            gissue(j + 1, (jj + 3) % 4)
            gwait(b)
            sissue(j, b)
            prefetch(j + 2, jj % 4)
        return carry

    lax.fori_loop(0, 30, quad, 0)
    swait(0)
    iwait(3)
    gissue(123, 3)
    gwait(2)
    sissue(122, 2)
    prefetch(124, 0)
    swait(1)
    iwait(0)
    gissue(124, 0)
    gwait(3)
    sissue(123, 3)
    swait(2)
    gwait(0)
    sissue(124, 0)
    swait(3)
    swait(0)
    plsc.subcore_barrier()
    pltpu.sync_copy(acc_sh.at[pl.ds(r0, RPS)], out_hbm.at[c, pl.ds(r0, RPS)])


def _sc_scatter(g, ei):
    idx_t = [pltpu.VMEM((EK,), jnp.int32) for _ in range(8)]
    row_t = [pltpu.VMEM((EK, D), jnp.float32) for _ in range(4)]
    sem_t = [pltpu.SemaphoreType.DMA for _ in range(12)]
    return pl.kernel(
        _scatter_body,
        out_type=jax.ShapeDtypeStruct((NC, NPAD, D), jnp.float32),
        mesh=_sc_mesh(),
        scratch_types=idx_t + row_t + sem_t + [
            pltpu.VMEM_SHARED((NPAD, D), jnp.float32),
        ],
    )(g, ei)


def _tc0_body(p_ref, dinv_ref):
    ones = jnp.ones((NC * NS, 1), jnp.float32)
    deg = lax.dot_general(p_ref[...], ones, (((0,), (0,)), ((), ()))) + 1.0
    dinv_ref[...] = jnp.broadcast_to(lax.rsqrt(deg), (BN, 8))


def _tc_stage0(parts):
    return pl.pallas_call(
        _tc0_body,
        grid=(NB,),
        in_specs=[pl.BlockSpec((NC * NS, BN), lambda i: (0, i))],
        out_specs=pl.BlockSpec((BN, 8), lambda i: (i, 0)),
        out_shape=jax.ShapeDtypeStruct((NPAD, 8), jnp.float32),
    )(parts)


def _tc1_body(x_ref, w_ref, dinv_ref, g_ref):
    dinv = dinv_ref[:, 0:1]
    h = jnp.dot(x_ref[...], w_ref[...], preferred_element_type=jnp.float32)
    g_ref[...] = h * dinv


def _tc_stage1(x, W1, dinv8):
    return pl.pallas_call(
        _tc1_body,
        grid=(NB,),
        in_specs=[
            pl.BlockSpec((BN, D), lambda i: (i, 0)),
            pl.BlockSpec((D, D), lambda i: (0, 0)),
            pl.BlockSpec((BN, 8), lambda i: (i, 0)),
        ],
        out_specs=pl.BlockSpec((BN, D), lambda i: (i, 0)),
        out_shape=jax.ShapeDtypeStruct((NPAD, D), jnp.float32),
    )(x, W1, dinv8)


def _tc2_body(acc_ref, g1_ref, dinv_ref, w2_ref, b1_ref, batch_ref,
              g2_ref, p1_ref, p1_acc):
    i = pl.program_id(0)
    dinv = dinv_ref[:, 0:1]
    esum = acc_ref[0] + acc_ref[1] - g1_ref[...]
    out1 = jax.nn.relu(esum * dinv + b1_ref[...])
    h2 = jnp.dot(out1, w2_ref[...], preferred_element_type=jnp.float32)
    g2_ref[...] = h2 * dinv
    onehot = (batch_ref[0] == lax.broadcasted_iota(jnp.int32, (G, BN), 0)
              ).astype(jnp.float32)
    part = jnp.dot(onehot, out1, preferred_element_type=jnp.float32)

    @pl.when(i == 0)
    def _():
        p1_acc[...] = jnp.zeros_like(p1_acc)

    p1_acc[...] += part

    @pl.when(i == NB - 1)
    def _():
        p1_ref[...] = p1_acc[...]


def _tc_stage2(acc1, g1, dinv8, W2, b1r, batch3):
    return pl.pallas_call(
        _tc2_body,
        grid=(NB,),
        in_specs=[
            pl.BlockSpec((NC, BN, D), lambda i: (0, i, 0)),
            pl.BlockSpec((BN, D), lambda i: (i, 0)),
            pl.BlockSpec((BN, 8), lambda i: (i, 0)),
            pl.BlockSpec((D, D), lambda i: (0, 0)),
            pl.BlockSpec((1, D), lambda i: (0, 0)),
            pl.BlockSpec((1, 1, BN), lambda i: (i, 0, 0)),
        ],
        out_specs=[
            pl.BlockSpec((BN, D), lambda i: (i, 0)),
            pl.BlockSpec((G, D), lambda i: (0, 0)),
        ],
        out_shape=[
            jax.ShapeDtypeStruct((NPAD, D), jnp.float32),
            jax.ShapeDtypeStruct((G, D), jnp.float32),
        ],
        scratch_shapes=[pltpu.VMEM((G, D), jnp.float32)],
    )(acc1, g1, dinv8, W2, b1r, batch3)


def _tc3_body(acc_ref, g2_ref, dinv_ref, b2_ref, batch_ref, p1_ref,
              wl1_ref, bl1_ref, wl2_ref, bl2_ref, h_ref, lsm_ref, p2_acc):
    i = pl.program_id(0)
    dinv = dinv_ref[:, 0:1]
    esum = acc_ref[0] + acc_ref[1] - g2_ref[...]
    out2 = jax.nn.relu(esum * dinv + b2_ref[...])
    onehot = (batch_ref[0] == lax.broadcasted_iota(jnp.int32, (G, BN), 0)
              ).astype(jnp.float32)
    part = jnp.dot(onehot, out2, preferred_element_type=jnp.float32)

    @pl.when(i == 0)
    def _():
        p2_acc[...] = jnp.zeros_like(p2_acc)

    p2_acc[...] += part

    @pl.when(i == NB - 1)
    def _():
        p = jnp.concatenate([p1_ref[...], p2_acc[...]], axis=1)
        h = jnp.dot(p, wl1_ref[...], preferred_element_type=jnp.float32)
        h = jax.nn.relu(h + bl1_ref[...])
        h = jnp.dot(h, wl2_ref[...], preferred_element_type=jnp.float32)
        h = h + bl2_ref[...]
        m = jnp.max(h, axis=1, keepdims=True)
        lse = jnp.log(jnp.sum(jnp.exp(h - m), axis=1, keepdims=True))
        h_ref[...] = h
        lsm_ref[...] = h - m - lse


def _tc_stage3(acc2, g2, dinv8, b2r, batch3, p1, Wl1, bl1r, Wl2, bl2r):
    return pl.pallas_call(
        _tc3_body,
        grid=(NB,),
        in_specs=[
            pl.BlockSpec((NC, BN, D), lambda i: (0, i, 0)),
            pl.BlockSpec((BN, D), lambda i: (i, 0)),
            pl.BlockSpec((BN, 8), lambda i: (i, 0)),
            pl.BlockSpec((1, D), lambda i: (0, 0)),
            pl.BlockSpec((1, 1, BN), lambda i: (i, 0, 0)),
            pl.BlockSpec((G, D), lambda i: (0, 0)),
            pl.BlockSpec((2 * D, 2 * D), lambda i: (0, 0)),
            pl.BlockSpec((1, 2 * D), lambda i: (0, 0)),
            pl.BlockSpec((2 * D, 10), lambda i: (0, 0)),
            pl.BlockSpec((1, 10), lambda i: (0, 0)),
        ],
        out_specs=[
            pl.BlockSpec((G, 10), lambda i: (0, 0)),
            pl.BlockSpec((G, 10), lambda i: (0, 0)),
        ],
        out_shape=[
            jax.ShapeDtypeStruct((G, 10), jnp.float32),
            jax.ShapeDtypeStruct((G, 10), jnp.float32),
        ],
        scratch_shapes=[pltpu.VMEM((G, D), jnp.float32)],
    )(acc2, g2, dinv8, b2r, batch3, p1, Wl1, bl1r, Wl2, bl2r)


def kernel(x, edge_index, batch, W1, b1, W2, b2, Wl1, bl1, Wl2, bl2):
    batchp = jnp.pad(batch, (0, NPAD - N), constant_values=G)
    batch3 = jnp.reshape(batchp, (NB, 1, BN))

    xp = jnp.pad(x, ((0, NPAD - N), (0, 0)))

    eif = jnp.reshape(edge_index, (2 * E,))
    parts = jnp.reshape(_sc_degree(eif), (NC * NS, NPAD))
    dinv8 = _tc_stage0(parts)
    g1 = _tc_stage1(xp, W1, dinv8)
    acc1 = _sc_scatter(g1, eif)
    g2, p1 = _tc_stage2(acc1, g1, dinv8, W2, jnp.reshape(b1, (1, D)), batch3)
    acc2 = _sc_scatter(g2, eif)
    h, lsm = _tc_stage3(acc2, g2, dinv8, jnp.reshape(b2, (1, D)), batch3, p1,
                        Wl1, jnp.reshape(bl1, (1, 2 * D)), Wl2,
                        jnp.reshape(bl2, (1, 10)))
    return (h, lsm)
